# baseline (speedup 1.0000x reference)
import functools

import jax
import jax.numpy as jnp
from jax import lax
from jax.experimental import pallas as pl
from jax.experimental.pallas import tpu as pltpu
from jax.experimental.pallas import tpu_sc as plsc

N_NODES = 10000
N_EDGES = 320000
D_FEAT = 128
LANES = 16
NUM_CORES = 2
NUM_SUBCORES = 16
NUM_TILES = NUM_CORES * NUM_SUBCORES
EDGES_PER_TILE = N_EDGES // NUM_TILES
CHUNK = 128
NCH = 80
EPT_PAD = NCH * CHUNK
PAD = EPT_PAD - EDGES_PER_TILE
ROWS_PER_TILE = 624
ZROWS = 48
SLOPE = 0.5


def _sc_body(row_hbm, col_hbm, val_hbm, emb_hbm, out_hbm,
             colv, row0, row1, val0, val1, rows0, rows1, zbuf, shared,
             gsem0, gsem1, isem0, isem1):
    c = lax.axis_index("c")
    s = lax.axis_index("s")
    wid = c * NUM_SUBCORES + s
    ebase = wid * EPT_PAD
    rows_b = (rows0, rows1)
    row_b = (row0, row1)
    val_b = (val0, val1)
    gsem_b = (gsem0, gsem1)
    isem_b = (isem0, isem1)

    pltpu.sync_copy(col_hbm.at[pl.ds(ebase, EPT_PAD)], colv)

    def _zero_z(i, _):
        for j in range(D_FEAT // LANES):
            zbuf[i, pl.ds(j * LANES, LANES)] = jnp.zeros((LANES,), jnp.float32)
        return 0
    lax.fori_loop(0, ZROWS, _zero_z, 0)
    rbase = s * ROWS_PER_TILE
    for t in range(ROWS_PER_TILE // ZROWS):
        pltpu.sync_copy(zbuf, shared.at[pl.ds(rbase + t * ZROWS, ZROWS)])
    @pl.when(s == NUM_SUBCORES - 1)
    def _zero_rem():
        pltpu.sync_copy(zbuf.at[pl.ds(0, N_NODES - NUM_SUBCORES * ROWS_PER_TILE)],
                        shared.at[pl.ds(NUM_SUBCORES * ROWS_PER_TILE,
                                        N_NODES - NUM_SUBCORES * ROWS_PER_TILE)])
    plsc.subcore_barrier()

    def _issue(k, b):
        pltpu.async_copy(emb_hbm.at[colv.at[pl.ds(k * CHUNK, CHUNK)]],
                         rows_b[b], gsem_b[b])
        pltpu.async_copy(row_hbm.at[pl.ds(ebase + k * CHUNK, CHUNK)],
                         row_b[b], isem_b[b])
        pltpu.async_copy(val_hbm.at[pl.ds(ebase + k * CHUNK, CHUNK)],
                         val_b[b], isem_b[b])

    def _wait(k, b):
        pltpu.make_async_copy(emb_hbm.at[colv.at[pl.ds(k * CHUNK, CHUNK)]],
                              rows_b[b], gsem_b[b]).wait()
        pltpu.make_async_copy(row_hbm.at[pl.ds(ebase + k * CHUNK, CHUNK)],
                              row_b[b], isem_b[b]).wait()
        pltpu.make_async_copy(val_hbm.at[pl.ds(ebase + k * CHUNK, CHUNK)],
                              val_b[b], isem_b[b]).wait()

    def _scale(b):
        rowsb = rows_b[b]
        valb = val_b[b]

        def _group(g, _):
            val16 = valb[pl.ds(g * LANES, LANES)]
            for e_loc in range(LANES):
                bvec = jnp.take_along_axis(
                    val16, jnp.full((LANES,), e_loc, jnp.int32), axis=0)
                e = g * LANES + e_loc
                for j in range(D_FEAT // LANES):
                    sl = pl.ds(j * LANES, LANES)
                    rowsb[e, sl] = rowsb[e, sl] * bvec
            return 0
        lax.fori_loop(0, CHUNK // LANES, _group, 0)

    _issue(0, 0)

    def _pair(i, _):
        for b in range(2):
            k = i * 2 + b

            @pl.when(k + 1 < NCH)
            def _prefetch():
                _issue(k + 1, 1 - b)
            _wait(k, b)
            _scale(b)
            pltpu.sync_copy(rows_b[b], shared.at[row_b[b]], add=True)
        return 0
    lax.fori_loop(0, NCH // 2, _pair, 0)
    plsc.subcore_barrier()

    pltpu.sync_copy(shared.at[pl.ds(rbase, ROWS_PER_TILE)],
                    out_hbm.at[c, pl.ds(rbase, ROWS_PER_TILE)])
    @pl.when(s == NUM_SUBCORES - 1)
    def _write_rem():
        r = NUM_SUBCORES * ROWS_PER_TILE
        pltpu.sync_copy(shared.at[pl.ds(r, N_NODES - r)],
                        out_hbm.at[c, pl.ds(r, N_NODES - r)])


@functools.partial(
    pl.kernel,
    out_type=jax.ShapeDtypeStruct((NUM_CORES, N_NODES, D_FEAT), jnp.float32),
    mesh=plsc.VectorSubcoreMesh(core_axis_name="c", subcore_axis_name="s"),
    scratch_types=[
        pltpu.VMEM((EPT_PAD,), jnp.int32),
        pltpu.VMEM((CHUNK,), jnp.int32),
        pltpu.VMEM((CHUNK,), jnp.int32),
        pltpu.VMEM((CHUNK,), jnp.float32),
        pltpu.VMEM((CHUNK,), jnp.float32),
        pltpu.VMEM((CHUNK, D_FEAT), jnp.float32),
        pltpu.VMEM((CHUNK, D_FEAT), jnp.float32),
        pltpu.VMEM((ZROWS, D_FEAT), jnp.float32),
        pltpu.VMEM_SHARED((N_NODES, D_FEAT), jnp.float32),
        pltpu.SemaphoreType.DMA,
        pltpu.SemaphoreType.DMA,
        pltpu.SemaphoreType.DMA,
        pltpu.SemaphoreType.DMA,
    ],
)
def _sc_spmm(row_hbm, col_hbm, val_hbm, emb_hbm, out_hbm, *scratch):
    _sc_body(row_hbm, col_hbm, val_hbm, emb_hbm, out_hbm, *scratch)


def _combine_body(p_ref, o_ref):
    x = p_ref[0] + p_ref[1]
    o_ref[...] = jnp.where(x >= 0, x, SLOPE * x)


def _combine(partials):
    blk = 1000
    return pl.pallas_call(
        _combine_body,
        grid=(N_NODES // blk,),
        in_specs=[pl.BlockSpec((NUM_CORES, blk, D_FEAT), lambda i: (0, i, 0))],
        out_specs=pl.BlockSpec((blk, D_FEAT), lambda i: (i, 0)),
        out_shape=jax.ShapeDtypeStruct((N_NODES, D_FEAT), jnp.float32),
    )(partials)


def kernel(adj_indices, adj_values, embeds):
    idx = adj_indices.astype(jnp.int32)
    pad2 = ((0, 0), (0, PAD))
    row1 = jnp.pad(idx[0].reshape(NUM_TILES, EDGES_PER_TILE), pad2).reshape(-1)
    col1 = jnp.pad(idx[1].reshape(NUM_TILES, EDGES_PER_TILE), pad2).reshape(-1)
    val1 = jnp.pad(adj_values.reshape(NUM_TILES, EDGES_PER_TILE), pad2).reshape(-1)
    partials = _sc_spmm(row1, col1, val1, embeds)
    return _combine(partials)
